# baseline (speedup 1.0000x reference)
import functools

import jax
import jax.numpy as jnp
from jax import lax
from jax.experimental import pallas as pl
from jax.experimental.pallas import tpu as pltpu
from jax.experimental.pallas import tpu_sc as plsc

NC = 2
NS = 16
EB = 256
BN = 2000


def _mesh():
    return plsc.VectorSubcoreMesh(core_axis_name="c", subcore_axis_name="s")


def _zero_fill(dst_sh, src_v, r0, rpt):
    nfull = rpt // EB
    rem = rpt % EB
    for i in range(nfull):
        pltpu.sync_copy(src_v, dst_sh.at[pl.ds(r0 + i * EB, EB)])
    if rem:
        pltpu.sync_copy(src_v.at[pl.ds(0, rem)],
                        dst_sh.at[pl.ds(r0 + nfull * EB, rem)])


def _make_deg_kernel(n_acc, nblk):
    rpt = n_acc // NS

    @functools.partial(
        pl.kernel,
        out_type=jax.ShapeDtypeStruct((NC, n_acc, 16), jnp.float32),
        mesh=_mesh(),
        compiler_params=pltpu.CompilerParams(use_tc_tiling_on_sc=False),
        scratch_types=[
            pltpu.VMEM_SHARED((n_acc, 16), jnp.float32),
            pltpu.VMEM((nblk, EB), jnp.int32),
            pltpu.VMEM((EB, 16), jnp.float32),
            pltpu.SemaphoreType.DMA,
        ],
    )
    def k(dstb_hbm, out_hbm, acc_sh, dst2d, ones_v, ssem):
        cid = lax.axis_index("c")
        sid = lax.axis_index("s")
        wid = sid * NC + cid
        zero = jnp.zeros((16,), jnp.float32)
        one = jnp.ones((16,), jnp.float32)

        @pl.loop(0, EB)
        def _(j):
            ones_v[j, :] = zero

        r0 = sid * rpt
        _zero_fill(acc_sh, ones_v, r0, rpt)

        @pl.loop(0, EB)
        def _(j):
            ones_v[j, :] = one

        pltpu.sync_copy(dstb_hbm.at[pl.ds(wid * nblk, nblk)], dst2d)
        plsc.subcore_barrier()

        @pl.loop(0, nblk // 4)
        def _(c):
            for jj in range(4):
                pltpu.async_copy(ones_v, acc_sh.at[dst2d.at[c * 4 + jj]],
                                 ssem, add=True)
            for jj in range(4):
                pltpu.make_async_copy(ones_v, acc_sh.at[dst2d.at[0]],
                                      ssem).wait()

        plsc.subcore_barrier()
        pltpu.sync_copy(acc_sh.at[pl.ds(r0, rpt)],
                        out_hbm.at[cid, pl.ds(r0, rpt)])

    return k


def _make_edge_kernel(n_acc, dx, nblk, gna):
    rpt = n_acc // NS
    n2 = nblk // 2

    scratch = [
        pltpu.VMEM_SHARED((n_acc, dx), jnp.float32),
        pltpu.VMEM((nblk, EB), jnp.int32),
        pltpu.VMEM((nblk, EB), jnp.int32),
        pltpu.VMEM((EB, dx), jnp.float32),
        pltpu.VMEM((EB, dx), jnp.float32),
        pltpu.SemaphoreType.DMA,
        pltpu.SemaphoreType.DMA,
        pltpu.SemaphoreType.DMA,
        pltpu.SemaphoreType.DMA,
    ]
    if gna:
        scratch += [
            pltpu.VMEM((EB, 16), jnp.float32),
            pltpu.VMEM((EB, 16), jnp.float32),
            pltpu.VMEM((16,), jnp.float32),
        ]

    @functools.partial(
        pl.kernel,
        out_type=jax.ShapeDtypeStruct((NC, n_acc, dx), jnp.float32),
        mesh=_mesh(),
        compiler_params=pltpu.CompilerParams(use_tc_tiling_on_sc=False),
        scratch_types=scratch,
    )
    def k(*args):
        if gna:
            (m_hbm, srcb_hbm, dstb_hbm, a_hbm, out_hbm,
             acc_sh, src2d, dst2d, ms0, ms1, gs0, gs1, ss0, ss1,
             md0, md1, a_v) = args
            rx, md = [ms0, ms1], [md0, md1]
        else:
            (hw_hbm, srcb_hbm, dstb_hbm, out_hbm,
             acc_sh, src2d, dst2d, rx0, rx1, gs0, gs1, ss0, ss1) = args
            rx = [rx0, rx1]
        gsem, ssem = [gs0, gs1], [ss0, ss1]
        cid = lax.axis_index("c")
        sid = lax.axis_index("s")
        wid = sid * NC + cid
        zero = jnp.zeros((16,), jnp.float32)

        @pl.loop(0, EB)
        def _(j):
            for t in range(dx // 16):
                rx[0][j, pl.ds(t * 16, 16)] = zero

        r0 = sid * rpt
        _zero_fill(acc_sh, rx[0], r0, rpt)
        if gna:
            pltpu.sync_copy(a_hbm, a_v)
        pltpu.sync_copy(srcb_hbm.at[pl.ds(wid * nblk, nblk)], src2d)
        pltpu.sync_copy(dstb_hbm.at[pl.ds(wid * nblk, nblk)], dst2d)
        plsc.subcore_barrier()

        def issue_gather(j, b):
            if gna:
                pltpu.async_copy(m_hbm.at[src2d.at[j]], rx[b], gsem[b])
                pltpu.async_copy(m_hbm.at[dst2d.at[j]], md[b], gsem[b])
            else:
                pltpu.async_copy(hw_hbm.at[src2d.at[j]], rx[b], gsem[b])

        def wait_gather(b):
            if gna:
                pltpu.make_async_copy(m_hbm.at[src2d.at[0]], rx[b],
                                      gsem[b]).wait()
                pltpu.make_async_copy(m_hbm.at[dst2d.at[0]], md[b],
                                      gsem[b]).wait()
            else:
                pltpu.make_async_copy(hw_hbm.at[src2d.at[0]], rx[b],
                                      gsem[b]).wait()

        def issue_scatter(j, b):
            pltpu.async_copy(rx[b], acc_sh.at[dst2d.at[j]], ssem[b],
                             add=True)

        def wait_scatter(b):
            pltpu.make_async_copy(rx[b], acc_sh.at[dst2d.at[0]],
                                  ssem[b]).wait()

        def compute(b):
            if not gna:
                return
            av = a_v[:]

            @plsc.parallel_loop(0, EB, unroll=8)
            def _(jj):
                msv = rx[b][jj, :]
                mdv = md[b][jj, :]
                t = (mdv - msv) * av
                rx[b][jj, :] = msv / (1.0 + jnp.exp(-t))

        issue_gather(0, 0)

        @pl.loop(0, n2)
        def _(i2):
            j0 = 2 * i2
            wait_gather(0)
            compute(0)

            @pl.when(i2 > 0)
            def _():
                wait_scatter(1)

            issue_scatter(j0, 0)
            issue_gather(j0 + 1, 1)
            wait_gather(1)
            compute(1)
            wait_scatter(0)
            issue_scatter(j0 + 1, 1)

            @pl.when(i2 < n2 - 1)
            def _():
                issue_gather(j0 + 2, 0)

        wait_scatter(1)
        plsc.subcore_barrier()
        pltpu.sync_copy(acc_sh.at[pl.ds(r0, rpt)],
                        out_hbm.at[cid, pl.ds(r0, rpt)])

    return k


def _rspec(d):
    return pl.BlockSpec((BN, d), lambda i: (i, 0))


def _pspec(d):
    return pl.BlockSpec((NC, BN, d), lambda i: (0, i, 0))


def _bspec(shape):
    return pl.BlockSpec(shape, lambda i: tuple(0 for _ in shape))


def _dinv_of(deg_ref):
    deg = deg_ref[0, :, 0] + deg_ref[1, :, 0] + 1.0
    return lax.rsqrt(deg)[:, None]


def _tc_pre_x(degp, x, w0):
    n, dxi = x.shape
    dxo = w0.shape[1]

    def body(deg_ref, x_ref, w0_ref, hw_ref):
        dinv = _dinv_of(deg_ref)
        hw_ref[...] = jnp.dot(x_ref[...], w0_ref[...],
                              preferred_element_type=jnp.float32) * dinv

    return pl.pallas_call(
        body,
        grid=(n // BN,),
        in_specs=[_pspec(16), _rspec(dxi), _bspec((dxi, dxo))],
        out_specs=_rspec(dxo),
        out_shape=jax.ShapeDtypeStruct((n, dxo), jnp.float32),
    )(degp, x, w0)


def _tc_pre_s(s, w2p, b2p, w1p, b1p):
    n = s.shape[0]

    def body(s_ref, w2_ref, b2_ref, w1_ref, b1_ref, m_ref, gw1_ref):
        sv = s_ref[...]
        m_ref[...] = jnp.dot(sv, w2_ref[...],
                             preferred_element_type=jnp.float32) + b2_ref[...]
        gw1_ref[...] = jnp.dot(sv, w1_ref[...],
                               preferred_element_type=jnp.float32) + b1_ref[...]

    return pl.pallas_call(
        body,
        grid=(n // BN,),
        in_specs=[_rspec(16), _bspec((16, 16)), _bspec((1, 16)),
                  _bspec((16, 16)), _bspec((1, 16))],
        out_specs=[_rspec(16), _rspec(16)],
        out_shape=[jax.ShapeDtypeStruct((n, 16), jnp.float32),
                   jax.ShapeDtypeStruct((n, 16), jnp.float32)],
    )(s, w2p, b2p, w1p, b1p)


def _tc_x(degp, accx, hwp, bxp, w):
    n, dprev = hwp.shape
    dxo = dprev if w is None else w.shape[1]

    def body(*refs):
        if w is None:
            deg_ref, ax_ref, hwp_ref, bx_ref, hw_ref = refs
        else:
            deg_ref, ax_ref, hwp_ref, bx_ref, w_ref, hw_ref = refs
        dinv = _dinv_of(deg_ref)
        h = jnp.maximum(
            dinv * (ax_ref[0] + ax_ref[1] + hwp_ref[...]) + bx_ref[...], 0.0)
        if w is None:
            hw_ref[...] = h * dinv
        else:
            hw_ref[...] = jnp.dot(h, w_ref[...],
                                  preferred_element_type=jnp.float32) * dinv

    ins = [degp, accx, hwp, bxp]
    specs = [_pspec(16), _pspec(dprev), _rspec(dprev), _bspec((1, dprev))]
    if w is not None:
        ins.append(w)
        specs.append(_bspec((dprev, dxo)))
    return pl.pallas_call(
        body,
        grid=(n // BN,),
        in_specs=specs,
        out_specs=_rspec(dxo),
        out_shape=jax.ShapeDtypeStruct((n, dxo), jnp.float32),
    )(*ins)


def _tc_s(accs, gw1p, w2p, b2p, w1p, b1p):
    n = gw1p.shape[0]

    def body(as_ref, gw1p_ref, w2_ref, b2_ref, w1_ref, b1_ref,
             m_ref, gw1_ref):
        g = jnp.maximum(gw1p_ref[...] + as_ref[0] + as_ref[1], 0.0)
        m_ref[...] = jnp.dot(g, w2_ref[...],
                             preferred_element_type=jnp.float32) + b2_ref[...]
        gw1_ref[...] = jnp.dot(g, w1_ref[...],
                               preferred_element_type=jnp.float32) + b1_ref[...]

    return pl.pallas_call(
        body,
        grid=(n // BN,),
        in_specs=[_pspec(16), _rspec(16), _bspec((16, 16)), _bspec((1, 16)),
                  _bspec((16, 16)), _bspec((1, 16))],
        out_specs=[_rspec(16), _rspec(16)],
        out_shape=[jax.ShapeDtypeStruct((n, 16), jnp.float32),
                   jax.ShapeDtypeStruct((n, 16), jnp.float32)],
    )(accs, gw1p, w2p, b2p, w1p, b1p)


def _tc_final_x(degp, accx, hwp, w3, bx3):
    n, dprev = hwp.shape
    dxo = w3.shape[1]

    def body(deg_ref, ax_ref, hwp_ref, w3_ref, bx_ref, xo_ref):
        dinv = _dinv_of(deg_ref)
        agg = dinv * (ax_ref[0] + ax_ref[1] + hwp_ref[...])
        xo_ref[...] = jnp.dot(agg, w3_ref[...],
                              preferred_element_type=jnp.float32) + bx_ref[...]

    return pl.pallas_call(
        body,
        grid=(n // BN,),
        in_specs=[_pspec(16), _pspec(dprev), _rspec(dprev),
                  _bspec((dprev, dxo)), _bspec((1, dxo))],
        out_specs=_rspec(dxo),
        out_shape=jax.ShapeDtypeStruct((n, dxo), jnp.float32),
    )(degp, accx, hwp, w3, bx3)


def _tc_final_s(accs, gw1p):
    n = gw1p.shape[0]

    def body(as_ref, gw1p_ref, so_ref):
        so_ref[...] = gw1p_ref[...] + as_ref[0] + as_ref[1]

    return pl.pallas_call(
        body,
        grid=(n // BN,),
        in_specs=[_pspec(16), _rspec(16)],
        out_specs=_rspec(16),
        out_shape=jax.ShapeDtypeStruct((n, 16), jnp.float32),
    )(accs, gw1p)


def _pad16(w):
    out = jnp.zeros((16, 16), jnp.float32)
    return out.at[: w.shape[0], : w.shape[1]].set(w)


def _padv(v):
    out = jnp.zeros((1, 16), jnp.float32)
    return out.at[0, : v.shape[0]].set(v)


def kernel(x, s, edge_index, gcn_params, gna_params):
    n = x.shape[0]
    e = edge_index.shape[1]
    nw = NC * NS
    src = edge_index[0].astype(jnp.int32)
    dst = edge_index[1].astype(jnp.int32)
    n_acc = -(-(n + 1) // (NS * 8)) * (NS * 8) + NS * 8 * 2
    nblk = -(-e // (nw * EB))
    nblk += nblk % 2
    e_pad = nblk * nw * EB
    pad = e_pad - e
    trash = n + (jnp.arange(pad, dtype=jnp.int32) % (n_acc - n))
    srcb = jnp.concatenate([src, jnp.arange(pad, dtype=jnp.int32) % n])
    dstb = jnp.concatenate([dst, trash])
    srcb = srcb.reshape(nw * nblk, EB)
    dstb = dstb.reshape(nw * nblk, EB)

    degp = _make_deg_kernel(n_acc, nblk)(dstb)

    w1ps, b1ps, w2ps, b2ps, aps = [], [], [], [], []
    for (w1, b1, w2, b2, a) in gna_params:
        w1ps.append(_pad16(w1))
        b1ps.append(_padv(b1))
        w2ps.append(_pad16(w2))
        b2ps.append(_padv(b2))
        aps.append(_padv(a)[0])
    bxs = [p[1][None, :] for p in gcn_params]

    gcn_edge = _make_edge_kernel(n_acc, 64, nblk, gna=False)
    gna_edge = _make_edge_kernel(n_acc, 16, nblk, gna=True)

    m, gw1 = _tc_pre_s(s, w2ps[0], b2ps[0], w1ps[0], b1ps[0])
    hw = _tc_pre_x(degp, x, gcn_params[0][0])
    nl = len(gcn_params)
    x_ = s_ = None
    for i in range(nl):
        accx = gcn_edge(hw, srcb, dstb)
        accs = gna_edge(m, srcb, dstb, aps[i])
        if i < nl - 1:
            wnext = gcn_params[i + 1][0] if i < nl - 2 else None
            hw = _tc_x(degp, accx, hw, bxs[i], wnext)
            m, gw1 = _tc_s(accs, gw1, w2ps[i + 1], b2ps[i + 1],
                           w1ps[i + 1], b1ps[i + 1])
        else:
            x_ = _tc_final_x(degp, accx, hw, gcn_params[i][0], bxs[i])
            s_ = _tc_final_s(accs, gw1)
    return (x_, s_)


# speedup vs baseline: 1.0795x; 1.0795x over previous
import functools

import jax
import jax.numpy as jnp
from jax import lax
from jax.experimental import pallas as pl
from jax.experimental.pallas import tpu as pltpu
from jax.experimental.pallas import tpu_sc as plsc

NC = 2
NS = 16
IB = 128
EB = 256
BN = 2000


def _mesh():
    return plsc.VectorSubcoreMesh(core_axis_name="c", subcore_axis_name="s")


def _zero_fill(dst_sh, src_v, r0, rpt):
    nfull = rpt // EB
    rem = rpt % EB
    for i in range(nfull):
        pltpu.sync_copy(src_v, dst_sh.at[pl.ds(r0 + i * EB, EB)])
    if rem:
        pltpu.sync_copy(src_v.at[pl.ds(0, rem)],
                        dst_sh.at[pl.ds(r0 + nfull * EB, rem)])


def _make_deg_kernel(n_acc, nblk):
    rpt = n_acc // NS

    @functools.partial(
        pl.kernel,
        out_type=jax.ShapeDtypeStruct((NC, n_acc, 16), jnp.float32),
        mesh=_mesh(),
        compiler_params=pltpu.CompilerParams(use_tc_tiling_on_sc=False),
        scratch_types=[
            pltpu.VMEM_SHARED((n_acc, 16), jnp.float32),
            pltpu.VMEM((nblk, EB), jnp.int32),
            pltpu.VMEM((EB, 16), jnp.float32),
            pltpu.SemaphoreType.DMA,
        ],
    )
    def k(dstb_hbm, out_hbm, acc_sh, dst2d, ones_v, ssem):
        cid = lax.axis_index("c")
        sid = lax.axis_index("s")
        wid = sid * NC + cid
        zero = jnp.zeros((16,), jnp.float32)
        one = jnp.ones((16,), jnp.float32)

        @pl.loop(0, EB)
        def _(j):
            ones_v[j, :] = zero

        r0 = sid * rpt
        _zero_fill(acc_sh, ones_v, r0, rpt)

        @pl.loop(0, EB)
        def _(j):
            ones_v[j, :] = one

        pltpu.sync_copy(dstb_hbm.at[pl.ds(wid * nblk, nblk)], dst2d)
        plsc.subcore_barrier()

        @pl.loop(0, nblk // 4)
        def _(c):
            for jj in range(4):
                pltpu.async_copy(ones_v, acc_sh.at[dst2d.at[c * 4 + jj]],
                                 ssem, add=True)
            for jj in range(4):
                pltpu.make_async_copy(ones_v, acc_sh.at[dst2d.at[0]],
                                      ssem).wait()

        plsc.subcore_barrier()
        pltpu.sync_copy(acc_sh.at[pl.ds(r0, rpt)],
                        out_hbm.at[cid, pl.ds(r0, rpt)])

    return k


def _make_edge_kernel(n_acc, dx, nblk, include_gna):
    rpt = n_acc // NS
    n2 = nblk // 2

    out_type = [jax.ShapeDtypeStruct((NC, n_acc, dx), jnp.float32)]
    scratch = [
        pltpu.VMEM_SHARED((n_acc, dx), jnp.float32),
        pltpu.VMEM((nblk, EB), jnp.int32),
        pltpu.VMEM((nblk, EB), jnp.int32),
        pltpu.VMEM((EB, dx), jnp.float32),
        pltpu.VMEM((EB, dx), jnp.float32),
        pltpu.SemaphoreType.DMA,
        pltpu.SemaphoreType.DMA,
        pltpu.SemaphoreType.DMA,
        pltpu.SemaphoreType.DMA,
    ]
    if include_gna:
        out_type.append(jax.ShapeDtypeStruct((NC, n_acc, 16), jnp.float32))
        scratch += [
            pltpu.VMEM_SHARED((n_acc, 16), jnp.float32),
            pltpu.VMEM((EB, 16), jnp.float32),
            pltpu.VMEM((EB, 16), jnp.float32),
            pltpu.VMEM((EB, 16), jnp.float32),
            pltpu.VMEM((EB, 16), jnp.float32),
            pltpu.VMEM((16,), jnp.float32),
        ]

    @functools.partial(
        pl.kernel,
        out_type=tuple(out_type) if include_gna else out_type[0],
        mesh=_mesh(),
        compiler_params=pltpu.CompilerParams(use_tc_tiling_on_sc=False),
        scratch_types=scratch,
    )
    def k(*args):
        if include_gna:
            (hw_hbm, m_hbm, srcb_hbm, dstb_hbm, a_hbm, outx_hbm, outs_hbm,
             accx_sh, src2d, dst2d, rx0, rx1, gs0, gs1, ss0, ss1,
             accs_sh, ms0, ms1, md0, md1, a_v) = args
            ms, md = [ms0, ms1], [md0, md1]
        else:
            (hw_hbm, srcb_hbm, dstb_hbm, outx_hbm,
             accx_sh, src2d, dst2d, rx0, rx1, gs0, gs1, ss0, ss1) = args
        rx, gsem, ssem = [rx0, rx1], [gs0, gs1], [ss0, ss1]
        cid = lax.axis_index("c")
        sid = lax.axis_index("s")
        wid = sid * NC + cid
        zero = jnp.zeros((16,), jnp.float32)

        @pl.loop(0, EB)
        def _(j):
            for t in range(dx // 16):
                rx0[j, pl.ds(t * 16, 16)] = zero
            if include_gna:
                ms0[j, :] = zero

        r0 = sid * rpt
        _zero_fill(accx_sh, rx0, r0, rpt)
        if include_gna:
            _zero_fill(accs_sh, ms0, r0, rpt)
            pltpu.sync_copy(a_hbm, a_v)
        pltpu.sync_copy(srcb_hbm.at[pl.ds(wid * nblk, nblk)], src2d)
        pltpu.sync_copy(dstb_hbm.at[pl.ds(wid * nblk, nblk)], dst2d)
        plsc.subcore_barrier()

        def issue_gather(j, b):
            pltpu.async_copy(hw_hbm.at[src2d.at[j]], rx[b], gsem[b])
            if include_gna:
                pltpu.async_copy(m_hbm.at[src2d.at[j]], ms[b], gsem[b])
                pltpu.async_copy(m_hbm.at[dst2d.at[j]], md[b], gsem[b])

        def wait_gather(b):
            pltpu.make_async_copy(hw_hbm.at[src2d.at[0]], rx[b],
                                  gsem[b]).wait()
            if include_gna:
                pltpu.make_async_copy(m_hbm.at[src2d.at[0]], ms[b],
                                      gsem[b]).wait()
                pltpu.make_async_copy(m_hbm.at[dst2d.at[0]], md[b],
                                      gsem[b]).wait()

        def issue_scatter(j, b):
            pltpu.async_copy(rx[b], accx_sh.at[dst2d.at[j]], ssem[b],
                             add=True)
            if include_gna:
                pltpu.async_copy(ms[b], accs_sh.at[dst2d.at[j]], ssem[b],
                                 add=True)

        def wait_scatter(b):
            pltpu.make_async_copy(rx[b], accx_sh.at[dst2d.at[0]],
                                  ssem[b]).wait()
            if include_gna:
                pltpu.make_async_copy(ms[b], accs_sh.at[dst2d.at[0]],
                                      ssem[b]).wait()

        def gna(b):
            if not include_gna:
                return
            av = a_v[:]

            @plsc.parallel_loop(0, EB, unroll=8)
            def _(jj):
                msv = ms[b][jj, :]
                mdv = md[b][jj, :]
                t = (mdv - msv) * av
                ms[b][jj, :] = msv / (1.0 + jnp.exp(-t))

        issue_gather(0, 0)

        @pl.loop(0, n2)
        def _(i2):
            j0 = 2 * i2
            wait_gather(0)
            gna(0)

            @pl.when(i2 > 0)
            def _():
                wait_scatter(1)

            issue_scatter(j0, 0)
            issue_gather(j0 + 1, 1)
            wait_gather(1)
            gna(1)
            wait_scatter(0)
            issue_scatter(j0 + 1, 1)

            @pl.when(i2 < n2 - 1)
            def _():
                issue_gather(j0 + 2, 0)

        wait_scatter(1)
        plsc.subcore_barrier()
        pltpu.sync_copy(accx_sh.at[pl.ds(r0, rpt)],
                        outx_hbm.at[cid, pl.ds(r0, rpt)])
        if include_gna:
            pltpu.sync_copy(accs_sh.at[pl.ds(r0, rpt)],
                            outs_hbm.at[cid, pl.ds(r0, rpt)])

    return k


def _rspec(d):
    return pl.BlockSpec((BN, d), lambda i: (i, 0))


def _pspec(d):
    return pl.BlockSpec((NC, BN, d), lambda i: (0, i, 0))


def _bspec(shape):
    return pl.BlockSpec(shape, lambda i: tuple(0 for _ in shape))


def _dinv_of(deg_ref):
    deg = deg_ref[0, :, 0] + deg_ref[1, :, 0] + 1.0
    return lax.rsqrt(deg)[:, None]


def _tc_pre(degp, x, s, w0, w2p, b2p, w1p, b1p):
    n, dxi = x.shape
    dxo = w0.shape[1]

    def body(deg_ref, x_ref, s_ref, w0_ref, w2_ref, b2_ref, w1_ref, b1_ref,
             hw_ref, m_ref, gw1_ref):
        dinv = _dinv_of(deg_ref)
        hw_ref[...] = jnp.dot(x_ref[...], w0_ref[...],
                              preferred_element_type=jnp.float32) * dinv
        sv = s_ref[...]
        m_ref[...] = jnp.dot(sv, w2_ref[...],
                             preferred_element_type=jnp.float32) + b2_ref[...]
        gw1_ref[...] = jnp.dot(sv, w1_ref[...],
                               preferred_element_type=jnp.float32) + b1_ref[...]

    return pl.pallas_call(
        body,
        grid=(n // BN,),
        in_specs=[_pspec(16), _rspec(dxi),
                  _rspec(16), _bspec((dxi, dxo)), _bspec((16, 16)),
                  _bspec((1, 16)), _bspec((16, 16)), _bspec((1, 16))],
        out_specs=[_rspec(dxo), _rspec(16), _rspec(16)],
        out_shape=[jax.ShapeDtypeStruct((n, dxo), jnp.float32),
                   jax.ShapeDtypeStruct((n, 16), jnp.float32),
                   jax.ShapeDtypeStruct((n, 16), jnp.float32)],
    )(degp, x, s, w0, w2p, b2p, w1p, b1p)


def _tc_mid(degp, accx, accs, hwp, gw1p, bxp, w, w2p, b2p, w1p, b1p):
    n, dprev = hwp.shape
    dxo = dprev if w is None else w.shape[1]

    def body(*refs):
        if w is None:
            (deg_ref, ax_ref, as_ref, hwp_ref, gw1p_ref, bx_ref,
             w2_ref, b2_ref, w1_ref, b1_ref, hw_ref, m_ref, gw1_ref) = refs
        else:
            (deg_ref, ax_ref, as_ref, hwp_ref, gw1p_ref, bx_ref, w_ref,
             w2_ref, b2_ref, w1_ref, b1_ref, hw_ref, m_ref, gw1_ref) = refs
        dinv = _dinv_of(deg_ref)
        h = jnp.maximum(
            dinv * (ax_ref[0] + ax_ref[1] + hwp_ref[...]) + bx_ref[...], 0.0)
        if w is None:
            hw_ref[...] = h * dinv
        else:
            hw_ref[...] = jnp.dot(h, w_ref[...],
                                  preferred_element_type=jnp.float32) * dinv
        g = jnp.maximum(gw1p_ref[...] + as_ref[0] + as_ref[1], 0.0)
        m_ref[...] = jnp.dot(g, w2_ref[...],
                             preferred_element_type=jnp.float32) + b2_ref[...]
        gw1_ref[...] = jnp.dot(g, w1_ref[...],
                               preferred_element_type=jnp.float32) + b1_ref[...]

    ins = [degp, accx, accs, hwp, gw1p, bxp]
    specs = [_pspec(16), _pspec(dprev), _pspec(16),
             _rspec(dprev), _rspec(16), _bspec((1, dprev))]
    if w is not None:
        ins.append(w)
        specs.append(_bspec((dprev, dxo)))
    ins += [w2p, b2p, w1p, b1p]
    specs += [_bspec((16, 16)), _bspec((1, 16)),
              _bspec((16, 16)), _bspec((1, 16))]
    return pl.pallas_call(
        body,
        grid=(n // BN,),
        in_specs=specs,
        out_specs=[_rspec(dxo), _rspec(16), _rspec(16)],
        out_shape=[jax.ShapeDtypeStruct((n, dxo), jnp.float32),
                   jax.ShapeDtypeStruct((n, 16), jnp.float32),
                   jax.ShapeDtypeStruct((n, 16), jnp.float32)],
    )(*ins)


def _tc_final(degp, accx, accs, hwp, gw1p, w3, bx3):
    n, dprev = hwp.shape
    dxo = w3.shape[1]

    def body(deg_ref, ax_ref, as_ref, hwp_ref, gw1p_ref, w3_ref, bx_ref,
             xo_ref, so_ref):
        dinv = _dinv_of(deg_ref)
        agg = dinv * (ax_ref[0] + ax_ref[1] + hwp_ref[...])
        xo_ref[...] = jnp.dot(agg, w3_ref[...],
                              preferred_element_type=jnp.float32) + bx_ref[...]
        so_ref[...] = gw1p_ref[...] + as_ref[0] + as_ref[1]

    return pl.pallas_call(
        body,
        grid=(n // BN,),
        in_specs=[_pspec(16), _pspec(dprev), _pspec(16),
                  _rspec(dprev), _rspec(16), _bspec((dprev, dxo)),
                  _bspec((1, dxo))],
        out_specs=[_rspec(dxo), _rspec(16)],
        out_shape=[jax.ShapeDtypeStruct((n, dxo), jnp.float32),
                   jax.ShapeDtypeStruct((n, 16), jnp.float32)],
    )(degp, accx, accs, hwp, gw1p, w3, bx3)


def _pad16(w):
    out = jnp.zeros((16, 16), jnp.float32)
    return out.at[: w.shape[0], : w.shape[1]].set(w)


def _padv(v):
    out = jnp.zeros((1, 16), jnp.float32)
    return out.at[0, : v.shape[0]].set(v)


def kernel(x, s, edge_index, gcn_params, gna_params):
    n = x.shape[0]
    e = edge_index.shape[1]
    nw = NC * NS
    src = edge_index[0].astype(jnp.int32)
    dst = edge_index[1].astype(jnp.int32)
    n_acc = -(-(n + 1) // (NS * 8)) * (NS * 8) + NS * 8 * 2
    nblk = -(-e // (nw * EB))
    nblk += nblk % 2
    e_pad = nblk * nw * EB
    pad = e_pad - e
    trash = n + (jnp.arange(pad, dtype=jnp.int32) % (n_acc - n))
    srcb = jnp.concatenate([src, jnp.arange(pad, dtype=jnp.int32) % n])
    dstb = jnp.concatenate([dst, trash])
    srcb = srcb.reshape(nw * nblk, EB)
    dstb = dstb.reshape(nw * nblk, EB)

    degp = _make_deg_kernel(n_acc, nblk)(dstb)

    w1ps, b1ps, w2ps, b2ps, aps = [], [], [], [], []
    for (w1, b1, w2, b2, a) in gna_params:
        w1ps.append(_pad16(w1))
        b1ps.append(_padv(b1))
        w2ps.append(_pad16(w2))
        b2ps.append(_padv(b2))
        aps.append(_padv(a)[0])
    bxs = [p[1][None, :] for p in gcn_params]

    hw, m, gw1 = _tc_pre(degp, x, s, gcn_params[0][0], w2ps[0], b2ps[0],
                         w1ps[0], b1ps[0])
    nl = len(gcn_params)
    x_ = s_ = None
    for i in range(nl):
        accx, accs = _make_edge_kernel(n_acc, hw.shape[1], nblk, True)(
            hw, m, srcb, dstb, aps[i])
        if i < nl - 2:
            hw, m, gw1 = _tc_mid(degp, accx, accs, hw, gw1, bxs[i],
                                 gcn_params[i + 1][0], w2ps[i + 1],
                                 b2ps[i + 1], w1ps[i + 1], b1ps[i + 1])
        elif i == nl - 2:
            hw, m, gw1 = _tc_mid(degp, accx, accs, hw, gw1, bxs[i],
                                 None, w2ps[i + 1],
                                 b2ps[i + 1], w1ps[i + 1], b1ps[i + 1])
        else:
            x_, s_ = _tc_final(degp, accx, accs, hw, gw1,
                               gcn_params[i][0], bxs[i])
    return (x_, s_)
